# Initial kernel scaffold; baseline (speedup 1.0000x reference)
import jax
import jax.numpy as jnp
from jax.experimental import pallas as pl


def kernel(x, edge_index, edge_attr, batch, params):
    raise NotImplementedError("write your pallas kernel here")


# speedup vs baseline: 10.5597x; 10.5597x over previous
import functools

import jax
import jax.numpy as jnp
from jax import lax
from jax.experimental import pallas as pl
from jax.experimental.pallas import tpu as pltpu
from jax.experimental.pallas import tpu_sc as plsc

_HID = 64
_HEADS = 4
_CH = 16
_NG = 256
_N = 50000
_E = 800000
_EPAD = 819200
_BN = 2000
_BE = 8192


def _matmul_body(x_ref, w_ref, b_ref, o_ref):
    o_ref[...] = (
        jnp.dot(x_ref[...], w_ref[...], preferred_element_type=jnp.float32)
        + b_ref[...]
    )


def _dense_proj(x, w, b, bn):
    n, k = x.shape
    m = w.shape[1]
    grid = n // bn
    return pl.pallas_call(
        _matmul_body,
        grid=(grid,),
        in_specs=[
            pl.BlockSpec((bn, k), lambda i: (i, 0)),
            pl.BlockSpec((k, m), lambda i: (0, 0)),
            pl.BlockSpec((1, m), lambda i: (0, 0)),
        ],
        out_specs=pl.BlockSpec((bn, m), lambda i: (i, 0)),
        out_shape=jax.ShapeDtypeStruct((n, m), jnp.float32),
    )(x, w, b.reshape(1, m))


def _eproj_body(ea_ref, w_ref, o_ref, s_ref):
    i = pl.program_id(0)
    o_ref[...] = jnp.dot(ea_ref[...], w_ref[...],
                         preferred_element_type=jnp.float32)

    @pl.when(i == 0)
    def _():
        s_ref[...] = jnp.zeros_like(s_ref)

    s_ref[...] += jnp.sum(ea_ref[...], axis=0, keepdims=True)


def _edge_proj(ea_pad, we):
    grid = _EPAD // _BE
    return pl.pallas_call(
        _eproj_body,
        grid=(grid,),
        in_specs=[
            pl.BlockSpec((_BE, 11), lambda i: (i, 0)),
            pl.BlockSpec((11, _HID), lambda i: (0, 0)),
        ],
        out_specs=[
            pl.BlockSpec((_BE, _HID), lambda i: (i, 0)),
            pl.BlockSpec((1, 11), lambda i: (0, 0)),
        ],
        out_shape=[
            jax.ShapeDtypeStruct((_EPAD, _HID), jnp.float32),
            jax.ShapeDtypeStruct((1, 11), jnp.float32),
        ],
    )(ea_pad, we)


def _edge_body(gs_ref, gd_ref, e_ref, att_ref, s_ref, st_ref, oa_ref, ob_ref):
    i = pl.program_id(0)
    gs = gs_ref[...]
    m = gs + gd_ref[...] + e_ref[...]
    m = jnp.where(m > 0, m, 0.2 * m)
    logit = jnp.dot(m * att_ref[...], s_ref[...],
                    preferred_element_type=jnp.float32)
    row = lax.broadcasted_iota(jnp.int32, (_BE, 1), 0) + i * _BE
    valid = (row < _E).astype(jnp.float32)
    ex = jnp.exp(logit) * valid
    w = gs * jnp.dot(ex, st_ref[...], preferred_element_type=jnp.float32)
    zero4 = jnp.zeros((_BE, 4), jnp.float32)
    oa_ref[...] = jnp.concatenate([w[:, :32], ex, zero4], axis=1)
    ob_ref[...] = jnp.concatenate([w[:, 32:], zero4, zero4], axis=1)


def _edge_stage(gs, gd, e, att_flat, sel, selt):
    grid = _EPAD // _BE
    return pl.pallas_call(
        _edge_body,
        grid=(grid,),
        in_specs=[
            pl.BlockSpec((_BE, _HID), lambda i: (i, 0)),
            pl.BlockSpec((_BE, _HID), lambda i: (i, 0)),
            pl.BlockSpec((_BE, _HID), lambda i: (i, 0)),
            pl.BlockSpec((1, _HID), lambda i: (0, 0)),
            pl.BlockSpec((_HID, _HEADS), lambda i: (0, 0)),
            pl.BlockSpec((_HEADS, _HID), lambda i: (0, 0)),
        ],
        out_specs=[
            pl.BlockSpec((_BE, 40), lambda i: (i, 0)),
            pl.BlockSpec((_BE, 40), lambda i: (i, 0)),
        ],
        out_shape=[
            jax.ShapeDtypeStruct((_EPAD, 40), jnp.float32),
            jax.ShapeDtypeStruct((_EPAD, 40), jnp.float32),
        ],
    )(gs, gd, e, att_flat, sel, selt)


def _node_body(acca_ref, accb_ref, xlr_ref, h_ref, easum_ref,
               we_ref, att_ref, s_ref, st_ref, bias_ref, g_ref, b_ref, o_ref):
    xlr = xlr_ref[...]
    xl = xlr[:, :_HID]
    emp = jnp.dot(easum_ref[...] * (1.0 / _E), we_ref[...],
                  preferred_element_type=jnp.float32)
    ms = xl + xlr[:, _HID:] + emp
    ms = jnp.where(ms > 0, ms, 0.2 * ms)
    exs = jnp.exp(jnp.dot(ms * att_ref[...], s_ref[...],
                          preferred_element_type=jnp.float32))
    acca = acca_ref[...]
    num = jnp.concatenate([acca[:, :32], accb_ref[:, :32]], axis=1)
    num = num + xl * jnp.dot(exs, st_ref[...],
                             preferred_element_type=jnp.float32)
    den = acca[:, 32:36] + exs + 1e-16
    o = num * jnp.dot(1.0 / den, st_ref[...],
                      preferred_element_type=jnp.float32)
    o = o + bias_ref[...]
    mu = jnp.mean(o, axis=-1, keepdims=True)
    v = jnp.mean((o - mu) ** 2, axis=-1, keepdims=True)
    o = (o - mu) * lax.rsqrt(v + 1e-5) * g_ref[...] + b_ref[...]
    o = 0.5 * o * (1.0 + lax.erf(o * 0.7071067811865476))
    o_ref[...] = h_ref[...] + o


def _node_stage(acc_a, acc_b, xlr, h, easum, we, att_flat, sel, selt,
                bias, g, b):
    grid = _N // _BN
    vec = lambda a: a.reshape(1, -1)
    return pl.pallas_call(
        _node_body,
        grid=(grid,),
        in_specs=[
            pl.BlockSpec((_BN, 40), lambda i: (i, 0)),
            pl.BlockSpec((_BN, 40), lambda i: (i, 0)),
            pl.BlockSpec((_BN, 2 * _HID), lambda i: (i, 0)),
            pl.BlockSpec((_BN, _HID), lambda i: (i, 0)),
            pl.BlockSpec((1, 11), lambda i: (0, 0)),
            pl.BlockSpec((11, _HID), lambda i: (0, 0)),
            pl.BlockSpec((1, _HID), lambda i: (0, 0)),
            pl.BlockSpec((_HID, _HEADS), lambda i: (0, 0)),
            pl.BlockSpec((_HEADS, _HID), lambda i: (0, 0)),
            pl.BlockSpec((1, _HID), lambda i: (0, 0)),
            pl.BlockSpec((1, _HID), lambda i: (0, 0)),
            pl.BlockSpec((1, _HID), lambda i: (0, 0)),
        ],
        out_specs=pl.BlockSpec((_BN, _HID), lambda i: (i, 0)),
        out_shape=jax.ShapeDtypeStruct((_N, _HID), jnp.float32),
    )(acc_a, acc_b, xlr, h, easum, we, vec(att_flat), sel, selt,
      vec(bias), vec(g), vec(b))


def _pool_body(b_ref, h_ref, s_ref, c_ref):
    i = pl.program_id(0)
    bb = b_ref[0, 0, :]
    gids = lax.broadcasted_iota(jnp.int32, (_NG, bb.shape[0]), 0)
    onehot = (gids == bb[None, :]).astype(jnp.float32)

    @pl.when(i == 0)
    def _():
        s_ref[...] = jnp.zeros_like(s_ref)
        c_ref[...] = jnp.zeros_like(c_ref)

    s_ref[...] += jnp.dot(onehot, h_ref[...],
                          preferred_element_type=jnp.float32)
    c_ref[...] += jnp.sum(onehot, axis=1, keepdims=True)


def _pool_stage(batch, h):
    bn = 2000
    grid = _N // bn
    b3 = batch.reshape(grid, 1, bn)
    return pl.pallas_call(
        _pool_body,
        grid=(grid,),
        in_specs=[
            pl.BlockSpec((1, 1, bn), lambda i: (i, 0, 0)),
            pl.BlockSpec((bn, _HID), lambda i: (i, 0)),
        ],
        out_specs=[
            pl.BlockSpec((_NG, _HID), lambda i: (0, 0)),
            pl.BlockSpec((_NG, 1), lambda i: (0, 0)),
        ],
        out_shape=[
            jax.ShapeDtypeStruct((_NG, _HID), jnp.float32),
            jax.ShapeDtypeStruct((_NG, 1), jnp.float32),
        ],
    )(b3, h)


def _head_body(s_ref, c_ref, w1_ref, b1_ref, w2_ref, b2_ref, o_ref):
    hg = s_ref[...] / jnp.maximum(c_ref[...], 1.0)
    z = jnp.dot(hg, w1_ref[...], preferred_element_type=jnp.float32) + b1_ref[...]
    z = 0.5 * z * (1.0 + lax.erf(z * 0.7071067811865476))
    z = jnp.dot(z, w2_ref[...], preferred_element_type=jnp.float32) + b2_ref[...]
    nrm = jnp.sqrt(jnp.sum(z * z, axis=-1, keepdims=True))
    o_ref[...] = z / jnp.maximum(nrm, 1e-12)


def _head_stage(sums, counts, w1, b1, w2, b2):
    return pl.pallas_call(
        _head_body,
        out_shape=jax.ShapeDtypeStruct((_NG, _HID), jnp.float32),
    )(sums, counts, w1, b1.reshape(1, -1), w2, b2.reshape(1, -1))


def _sc_gather(src, dst, xl, xr):
    gs = jnp.take(xl, src, axis=0)
    gd = jnp.take(xr, dst, axis=0)
    return gs, gd


def _sc_scatter(dst, wa, wb):
    acc_a = jax.ops.segment_sum(wa, dst, num_segments=_N)
    acc_b = jax.ops.segment_sum(wb, dst, num_segments=_N)
    return acc_a, acc_b


def kernel(x, edge_index, edge_attr, batch, params):
    f32 = jnp.float32
    src = edge_index[0].astype(jnp.int32)
    dst = edge_index[1].astype(jnp.int32)
    pad = _EPAD - _E
    src = jnp.pad(src, (0, pad))
    dst = jnp.pad(dst, (0, pad))
    ea = jnp.pad(edge_attr.astype(f32), ((0, pad), (0, 0)))

    sel = jnp.repeat(jnp.eye(_HEADS, dtype=f32), _CH, axis=0)
    selt = sel.T

    h = _dense_proj(x.astype(f32), params['Wp'], params['bp'], _BN)

    for lp in params['layers']:
        wlr = jnp.concatenate([lp['Wl'], lp['Wr']], axis=1)
        blr = jnp.concatenate([lp['bl'], lp['br']])
        xlr = _dense_proj(h, wlr, blr, _BN)
        e, easum = _edge_proj(ea, lp['We'])
        gs, gd = _sc_gather(src, dst, xlr[:, :_HID], xlr[:, _HID:])
        att_flat = lp['att'].reshape(1, _HID)
        wa, wb = _edge_stage(gs, gd, e, att_flat, sel, selt)
        acc_a, acc_b = _sc_scatter(dst, wa, wb)
        h = _node_stage(acc_a, acc_b, xlr, h, easum, lp['We'],
                        lp['att'].reshape(-1), sel, selt,
                        lp['bias'], lp['g'], lp['b'])

    sums, counts = _pool_stage(batch.astype(jnp.int32), h)
    return _head_stage(sums, counts, params['W1'], params['b1'],
                       params['W2'], params['b2'])


# speedup vs baseline: 14.9181x; 1.4127x over previous
import functools

import jax
import jax.numpy as jnp
from jax import lax
from jax.experimental import pallas as pl
from jax.experimental.pallas import tpu as pltpu
from jax.experimental.pallas import tpu_sc as plsc

_HID = 64
_HEADS = 4
_CH = 16
_NG = 256
_N = 50000
_E = 800000
_EPAD = 819200
_BN = 2000
_BE = 8192


def _matmul_body(x_ref, w_ref, b_ref, o_ref):
    o_ref[...] = (
        jnp.dot(x_ref[...], w_ref[...], preferred_element_type=jnp.float32)
        + b_ref[...]
    )


def _dense_proj(x, w, b, bn):
    n, k = x.shape
    m = w.shape[1]
    grid = n // bn
    return pl.pallas_call(
        _matmul_body,
        grid=(grid,),
        in_specs=[
            pl.BlockSpec((bn, k), lambda i: (i, 0)),
            pl.BlockSpec((k, m), lambda i: (0, 0)),
            pl.BlockSpec((1, m), lambda i: (0, 0)),
        ],
        out_specs=pl.BlockSpec((bn, m), lambda i: (i, 0)),
        out_shape=jax.ShapeDtypeStruct((n, m), jnp.float32),
    )(x, w, b.reshape(1, m))


def _eproj_body(ea_ref, w_ref, o_ref, s_ref):
    i = pl.program_id(0)
    o_ref[...] = jnp.dot(ea_ref[...], w_ref[...],
                         preferred_element_type=jnp.float32)

    @pl.when(i == 0)
    def _():
        s_ref[...] = jnp.zeros_like(s_ref)

    s_ref[...] += jnp.sum(ea_ref[...], axis=0, keepdims=True)


def _edge_proj(ea_pad, we):
    grid = _EPAD // _BE
    return pl.pallas_call(
        _eproj_body,
        grid=(grid,),
        in_specs=[
            pl.BlockSpec((_BE, 11), lambda i: (i, 0)),
            pl.BlockSpec((11, _HID), lambda i: (0, 0)),
        ],
        out_specs=[
            pl.BlockSpec((_BE, _HID), lambda i: (i, 0)),
            pl.BlockSpec((1, 11), lambda i: (0, 0)),
        ],
        out_shape=[
            jax.ShapeDtypeStruct((_EPAD, _HID), jnp.float32),
            jax.ShapeDtypeStruct((1, 11), jnp.float32),
        ],
    )(ea_pad, we)


def _edge_body(gs_ref, gd_ref, e_ref, att_ref, s_ref, st_ref, oa_ref):
    i = pl.program_id(0)
    gs = gs_ref[:, :_HID]
    m = gs + gd_ref[:, _HID:] + e_ref[...]
    m = jnp.where(m > 0, m, 0.2 * m)
    logit = jnp.dot(m * att_ref[...], s_ref[...],
                    preferred_element_type=jnp.float32)
    row = lax.broadcasted_iota(jnp.int32, (_BE, 1), 0) + i * _BE
    valid = (row < _E).astype(jnp.float32)
    ex = jnp.exp(logit) * valid
    w = gs * jnp.dot(ex, st_ref[...], preferred_element_type=jnp.float32)
    zero4 = jnp.zeros((_BE, 4), jnp.float32)
    oa_ref[0] = jnp.concatenate([w[:, :32], ex, zero4], axis=1)
    oa_ref[1] = jnp.concatenate([w[:, 32:], zero4, zero4], axis=1)


def _edge_stage(gs, gd, e, att_flat, sel, selt):
    grid = _EPAD // _BE
    return pl.pallas_call(
        _edge_body,
        grid=(grid,),
        in_specs=[
            pl.BlockSpec((_BE, 2 * _HID), lambda i: (i, 0)),
            pl.BlockSpec((_BE, 2 * _HID), lambda i: (i, 0)),
            pl.BlockSpec((_BE, _HID), lambda i: (i, 0)),
            pl.BlockSpec((1, _HID), lambda i: (0, 0)),
            pl.BlockSpec((_HID, _HEADS), lambda i: (0, 0)),
            pl.BlockSpec((_HEADS, _HID), lambda i: (0, 0)),
        ],
        out_specs=pl.BlockSpec((2, _BE, 40), lambda i: (0, i, 0)),
        out_shape=jax.ShapeDtypeStruct((2, _EPAD, 40), jnp.float32),
    )(gs, gd, e, att_flat, sel, selt)


def _node_body(acca_ref, accb_ref, xlr_ref, h_ref, easum_ref,
               we_ref, att_ref, s_ref, st_ref, bias_ref, g_ref, b_ref, o_ref):
    xl = xlr_ref[:, :_HID]
    emp = jnp.dot(easum_ref[...] * (1.0 / _E), we_ref[...],
                  preferred_element_type=jnp.float32)
    ms = xl + xlr_ref[:, _HID:] + emp
    ms = jnp.where(ms > 0, ms, 0.2 * ms)
    exs = jnp.exp(jnp.dot(ms * att_ref[...], s_ref[...],
                          preferred_element_type=jnp.float32))
    acca = acca_ref[...]
    num = jnp.concatenate([acca[:, :32], accb_ref[:, :32]], axis=1)
    num = num + xl * jnp.dot(exs, st_ref[...],
                             preferred_element_type=jnp.float32)
    den = acca[:, 32:36] + exs + 1e-16
    o = num * jnp.dot(1.0 / den, st_ref[...],
                      preferred_element_type=jnp.float32)
    o = o + bias_ref[...]
    mu = jnp.mean(o, axis=-1, keepdims=True)
    v = jnp.mean((o - mu) ** 2, axis=-1, keepdims=True)
    o = (o - mu) * lax.rsqrt(v + 1e-5) * g_ref[...] + b_ref[...]
    o = 0.5 * o * (1.0 + lax.erf(o * 0.7071067811865476))
    o_ref[...] = h_ref[...] + o


def _node_stage(acc_a, acc_b, xlr, h, easum, we, att_flat, sel, selt,
                bias, g, b):
    grid = _N // _BN
    vec = lambda a: a.reshape(1, -1)
    return pl.pallas_call(
        _node_body,
        grid=(grid,),
        in_specs=[
            pl.BlockSpec((_BN, 40), lambda i: (i, 0)),
            pl.BlockSpec((_BN, 40), lambda i: (i, 0)),
            pl.BlockSpec((_BN, 2 * _HID), lambda i: (i, 0)),
            pl.BlockSpec((_BN, _HID), lambda i: (i, 0)),
            pl.BlockSpec((1, 11), lambda i: (0, 0)),
            pl.BlockSpec((11, _HID), lambda i: (0, 0)),
            pl.BlockSpec((1, _HID), lambda i: (0, 0)),
            pl.BlockSpec((_HID, _HEADS), lambda i: (0, 0)),
            pl.BlockSpec((_HEADS, _HID), lambda i: (0, 0)),
            pl.BlockSpec((1, _HID), lambda i: (0, 0)),
            pl.BlockSpec((1, _HID), lambda i: (0, 0)),
            pl.BlockSpec((1, _HID), lambda i: (0, 0)),
        ],
        out_specs=pl.BlockSpec((_BN, _HID), lambda i: (i, 0)),
        out_shape=jax.ShapeDtypeStruct((_N, _HID), jnp.float32),
    )(acc_a, acc_b, xlr, h, easum, we, vec(att_flat), sel, selt,
      vec(bias), vec(g), vec(b))


def _pool_body(b_ref, h_ref, s_ref, c_ref):
    i = pl.program_id(0)
    bb = b_ref[0, 0, :]
    gids = lax.broadcasted_iota(jnp.int32, (_NG, bb.shape[0]), 0)
    onehot = (gids == bb[None, :]).astype(jnp.float32)

    @pl.when(i == 0)
    def _():
        s_ref[...] = jnp.zeros_like(s_ref)
        c_ref[...] = jnp.zeros_like(c_ref)

    s_ref[...] += jnp.dot(onehot, h_ref[...],
                          preferred_element_type=jnp.float32)
    c_ref[...] += jnp.sum(onehot, axis=1, keepdims=True)


def _pool_stage(batch, h):
    bn = 2000
    grid = _N // bn
    b3 = batch.reshape(grid, 1, bn)
    return pl.pallas_call(
        _pool_body,
        grid=(grid,),
        in_specs=[
            pl.BlockSpec((1, 1, bn), lambda i: (i, 0, 0)),
            pl.BlockSpec((bn, _HID), lambda i: (i, 0)),
        ],
        out_specs=[
            pl.BlockSpec((_NG, _HID), lambda i: (0, 0)),
            pl.BlockSpec((_NG, 1), lambda i: (0, 0)),
        ],
        out_shape=[
            jax.ShapeDtypeStruct((_NG, _HID), jnp.float32),
            jax.ShapeDtypeStruct((_NG, 1), jnp.float32),
        ],
    )(b3, h)


def _head_body(s_ref, c_ref, w1_ref, b1_ref, w2_ref, b2_ref, o_ref):
    hg = s_ref[...] / jnp.maximum(c_ref[...], 1.0)
    z = jnp.dot(hg, w1_ref[...], preferred_element_type=jnp.float32) + b1_ref[...]
    z = 0.5 * z * (1.0 + lax.erf(z * 0.7071067811865476))
    z = jnp.dot(z, w2_ref[...], preferred_element_type=jnp.float32) + b2_ref[...]
    nrm = jnp.sqrt(jnp.sum(z * z, axis=-1, keepdims=True))
    o_ref[...] = z / jnp.maximum(nrm, 1e-12)


def _head_stage(sums, counts, w1, b1, w2, b2):
    return pl.pallas_call(
        _head_body,
        out_shape=jax.ShapeDtypeStruct((_NG, _HID), jnp.float32),
    )(sums, counts, w1, b1.reshape(1, -1), w2, b2.reshape(1, -1))


_NC = 2
_NS = 16
_NW = _NC * _NS
_GK = 2
_GCHUNK = _GK * 128
_GCH = _EPAD // (_NW * _GCHUNK)
_SK = 4
_SCHUNK = _SK * 128
_SCH = _EPAD // (_NS * _SCHUNK)
_NR = 12544
_NPASS = 4
_NACC = _NR * _NPASS
_RPT = _NR // _NS
_WCH = 56


def _sc_gather(srcg, dstg, xlr):
    mesh = plsc.VectorSubcoreMesh(core_axis_name="c", subcore_axis_name="s")

    @functools.partial(
        pl.kernel,
        mesh=mesh,
        out_type=[
            jax.ShapeDtypeStruct((_EPAD, 2 * _HID), jnp.float32),
            jax.ShapeDtypeStruct((_EPAD, 2 * _HID), jnp.float32),
        ],
        scratch_types=[
            pltpu.VMEM((_GK, 128), jnp.int32),
            pltpu.VMEM((_GK, 128), jnp.int32),
            pltpu.VMEM((_GCHUNK, 2 * _HID), jnp.float32),
            pltpu.VMEM((_GCHUNK, 2 * _HID), jnp.float32),
            pltpu.SemaphoreType.DMA,
        ],
    )
    def gather_k(src_hbm, dst_hbm, t_hbm, gs_hbm, gd_hbm,
                 idxs_v, idxd_v, gs_v, gd_v, sem):
        wid = lax.axis_index("s") * _NC + lax.axis_index("c")

        def body(c, carry):
            chunk = wid * _GCH + c
            base = chunk * _GCHUNK
            pltpu.sync_copy(src_hbm.at[chunk], idxs_v)
            pltpu.sync_copy(dst_hbm.at[chunk], idxd_v)
            handles = []
            for j in range(_GK):
                handles.append(pltpu.async_copy(
                    t_hbm.at[idxs_v.at[j]],
                    gs_v.at[pl.ds(j * 128, 128)], sem))
                handles.append(pltpu.async_copy(
                    t_hbm.at[idxd_v.at[j]],
                    gd_v.at[pl.ds(j * 128, 128)], sem))
            for h in handles:
                h.wait()
            pltpu.sync_copy(gs_v, gs_hbm.at[pl.ds(base, _GCHUNK)])
            pltpu.sync_copy(gd_v, gd_hbm.at[pl.ds(base, _GCHUNK)])
            return carry

        lax.fori_loop(0, _GCH, body, 0)

    return gather_k(srcg, dstg, xlr)


def _sc_scatter_pass(p, dstv, wab, zeros_rows):
    mesh = plsc.VectorSubcoreMesh(core_axis_name="c", subcore_axis_name="s")
    lo = p * _NR

    @functools.partial(
        pl.kernel,
        mesh=mesh,
        out_type=jax.ShapeDtypeStruct((2, _NR, 40), jnp.float32),
        scratch_types=[
            pltpu.VMEM((_SK, 128), jnp.int32),
            pltpu.VMEM((_SK, 128), jnp.int32),
            pltpu.VMEM((_SCHUNK, 40), jnp.float32),
            pltpu.VMEM((_WCH, 40), jnp.float32),
            pltpu.VMEM_SHARED((_NR + 8, 40), jnp.float32),
            pltpu.SemaphoreType.DMA,
        ],
    )
    def scatter_k(dst_hbm, wab_hbm, z_hbm, acc_hbm,
                  idx_v, idx2_v, w_v, o_v, shared, sem):
        cid = lax.axis_index("c")
        sid = lax.axis_index("s")

        pltpu.sync_copy(z_hbm, shared.at[pl.ds(sid * _RPT, _RPT)])
        plsc.subcore_barrier()

        def body(c, carry):
            chunk = sid * _SCH + c
            base = chunk * _SCHUNK
            pltpu.sync_copy(dst_hbm.at[chunk], idx_v)
            pltpu.sync_copy(wab_hbm.at[cid, pl.ds(base, _SCHUNK)], w_v)
            for j in range(_SK):
                for k in range(8):
                    iv = idx_v[j, pl.ds(k * 16, 16)] - lo
                    ok = (iv >= 0) & (iv < _NR)
                    idx2_v[j, pl.ds(k * 16, 16)] = jnp.where(ok, iv, _NR)
                pltpu.sync_copy(
                    w_v.at[pl.ds(j * 128, 128)],
                    shared.at[idx2_v.at[j]], add=True)
            return carry

        lax.fori_loop(0, _SCH, body, 0)
        plsc.subcore_barrier()

        def wbody(r, carry):
            off = sid * _RPT + r * _WCH
            pltpu.sync_copy(shared.at[pl.ds(off, _WCH)], o_v)
            pltpu.sync_copy(o_v, acc_hbm.at[cid, pl.ds(off, _WCH)])
            return carry

        lax.fori_loop(0, _RPT // _WCH, wbody, 0)

    return scatter_k(dstv, wab, zeros_rows)


def _sc_scatter(dstv, wab, zeros_rows):
    accas, accbs = [], []
    for p in range(_NPASS):
        acc_p = _sc_scatter_pass(p, dstv, wab, zeros_rows)
        accas.append(acc_p[0])
        accbs.append(acc_p[1])
    return jnp.concatenate(accas, axis=0), jnp.concatenate(accbs, axis=0)


def kernel(x, edge_index, edge_attr, batch, params):
    f32 = jnp.float32
    src = edge_index[0].astype(jnp.int32)
    dst = edge_index[1].astype(jnp.int32)
    pad = _EPAD - _E
    src = jnp.pad(src, (0, pad))
    dst = jnp.pad(dst, (0, pad))
    ea = jnp.pad(edge_attr.astype(f32), ((0, pad), (0, 0)))

    sel = jnp.repeat(jnp.eye(_HEADS, dtype=f32), _CH, axis=0)
    selt = sel.T

    h = _dense_proj(x.astype(f32), params['Wp'], params['bp'], _BN)
    srcg = src.reshape(-1, _GK, 128)
    dstg = dst.reshape(-1, _GK, 128)
    dsts = dst.reshape(-1, _SK, 128)
    zeros_rows = jnp.zeros((_RPT, 40), f32)

    for lp in params['layers']:
        wlr = jnp.concatenate([lp['Wl'], lp['Wr']], axis=1)
        blr = jnp.concatenate([lp['bl'], lp['br']])
        xlr = _dense_proj(h, wlr, blr, _BN)
        e, easum = _edge_proj(ea, lp['We'])
        gs, gd = _sc_gather(srcg, dstg, xlr)
        att_flat = lp['att'].reshape(1, _HID)
        wab = _edge_stage(gs, gd, e, att_flat, sel, selt)
        acc_a = jax.ops.segment_sum(wab[0], dst, num_segments=_NACC)
        acc_b = jax.ops.segment_sum(wab[1], dst, num_segments=_NACC)
        h = _node_stage(acc_a, acc_b, xlr, h, easum, lp['We'],
                        lp['att'].reshape(-1), sel, selt,
                        lp['bias'], lp['g'], lp['b'])

    sums, counts = _pool_stage(batch.astype(jnp.int32), h)
    return _head_stage(sums, counts, params['W1'], params['b1'],
                       params['W2'], params['b2'])
